# Initial kernel scaffold; baseline (speedup 1.0000x reference)
import jax
import jax.numpy as jnp
from jax.experimental import pallas as pl


def kernel(x, edge_index, W1, b1, W2, b2):
    raise NotImplementedError("write your pallas kernel here")


# speedup vs baseline: 15.5459x; 15.5459x over previous
import functools

import jax
import jax.numpy as jnp
from jax import lax
from jax.experimental import pallas as pl
from jax.experimental.pallas import tpu as pltpu
from jax.experimental.pallas import tpu_sc as plsc

N_NODES = 10000
D = 128
N_EDGES = 320000

NC = 2
NS = 16
NW = NC * NS
EPT = N_EDGES // NW
CHUNK = 80
NCHUNK = EPT // CHUNK
IDX_BLK = 5
OUTER = NCHUNK // IDX_BLK
N_PAD = 10240
RPT = N_PAD // NS
SLAB = 64
DEG_LANES = 128

_MESH = plsc.VectorSubcoreMesh(core_axis_name="c", subcore_axis_name="s")


def _zero_rows(buf, nrows, ncols):
    @pl.loop(0, nrows)
    def _(r):
        @pl.loop(0, ncols, step=16)
        def _(cc):
            buf[r, pl.ds(cc, 16)] = jnp.zeros((16,), jnp.float32)


def _sc_indeg(dst3d, zeros16):
    @functools.partial(
        pl.kernel,
        mesh=_MESH,
        out_type=jax.ShapeDtypeStruct((NC, N_PAD, DEG_LANES), jnp.float32),
        scratch_types=[
            pltpu.VMEM((IDX_BLK, CHUNK), jnp.int32),
            pltpu.VMEM((CHUNK, DEG_LANES), jnp.float32),
            pltpu.VMEM_SHARED((N_PAD, DEG_LANES), jnp.float32),
        ],
    )
    def k(dst_hbm, z_hbm, out_hbm, idx_v, ones_v, acc_s):
        c = lax.axis_index("c")
        s = lax.axis_index("s")
        wid = c * NS + s

        @pl.loop(0, CHUNK)
        def _(r):
            ones_v[r, :] = jnp.ones((DEG_LANES,), jnp.float32)

        for k in range(NS):
            @pl.when(s == k)
            def _(k=k):
                pltpu.sync_copy(z_hbm.at[pl.ds(k * RPT, RPT)],
                                acc_s.at[pl.ds(k * RPT, RPT)])

        plsc.subcore_barrier()

        @pl.loop(0, OUTER)
        def _(t):
            pltpu.sync_copy(dst_hbm.at[wid, t], idx_v)

            @pl.loop(0, IDX_BLK)
            def _(r):
                pltpu.sync_copy(ones_v, acc_s.at[idx_v.at[r]], add=True)

        plsc.subcore_barrier()

        for k in range(NS):
            @pl.when(s == k)
            def _(k=k):
                pltpu.sync_copy(acc_s.at[pl.ds(k * RPT, RPT)],
                                out_hbm.at[c, pl.ds(k * RPT, RPT)])

    return k(dst3d, zeros16)


def _sc_agg(y, src4d, dst4d, zrows):
    @functools.partial(
        pl.kernel,
        mesh=_MESH,
        out_type=jax.ShapeDtypeStruct((NC, N_PAD, D), jnp.float32),
        scratch_types=[
            pltpu.VMEM((IDX_BLK, CHUNK), jnp.int32),
            pltpu.VMEM((IDX_BLK, CHUNK), jnp.int32),
            pltpu.VMEM((CHUNK, D), jnp.float32),
            pltpu.VMEM_SHARED((N_PAD, D), jnp.float32),
        ],
    )
    def k(y_hbm, src_hbm, dst_hbm, z_hbm, out_hbm, src_v, dst_v, rows_v, acc_s):
        c = lax.axis_index("c")
        s = lax.axis_index("s")
        wid = c * NS + s

        for k in range(NS):
            @pl.when(s == k)
            def _(k=k):
                pltpu.sync_copy(z_hbm.at[pl.ds(k * RPT, RPT)],
                                acc_s.at[pl.ds(k * RPT, RPT)])

        plsc.subcore_barrier()

        @pl.loop(0, OUTER)
        def _(t):
            pltpu.sync_copy(src_hbm.at[wid, t], src_v)
            pltpu.sync_copy(dst_hbm.at[wid, t], dst_v)

            @pl.loop(0, IDX_BLK)
            def _(r):
                pltpu.sync_copy(y_hbm.at[src_v.at[r]], rows_v)
                pltpu.sync_copy(rows_v, acc_s.at[dst_v.at[r]], add=True)

        plsc.subcore_barrier()

        for k in range(NS):
            @pl.when(s == k)
            def _(k=k):
                pltpu.sync_copy(acc_s.at[pl.ds(k * RPT, RPT)],
                                out_hbm.at[c, pl.ds(k * RPT, RPT)])

    return k(y, src4d, dst4d, zrows)


BR = 400


def _dis_block(deg_ref):
    degsum = deg_ref[0] + deg_ref[1]
    return lax.rsqrt(1.0 + degsum[:, :1])


def _tc_y1(x, W1, deg):
    def body(x_ref, w_ref, deg_ref, y_ref):
        dis = _dis_block(deg_ref)
        xw = jnp.dot(x_ref[...], w_ref[...], preferred_element_type=jnp.float32)
        y_ref[...] = xw * dis

    return pl.pallas_call(
        body,
        grid=(N_NODES // BR,),
        in_specs=[
            pl.BlockSpec((BR, D), lambda i: (i, 0)),
            pl.BlockSpec((D, D), lambda i: (0, 0)),
            pl.BlockSpec((2, BR, DEG_LANES), lambda i: (0, i, 0)),
        ],
        out_specs=pl.BlockSpec((BR, D), lambda i: (i, 0)),
        out_shape=jax.ShapeDtypeStruct((N_NODES, D), jnp.float32),
    )(x, W1, deg)


def _tc_mid(agg, y1, deg, W2, b1):
    def body(agg_ref, y1_ref, deg_ref, w_ref, b_ref, y2_ref):
        dis = _dis_block(deg_ref)
        t = (agg_ref[0] + agg_ref[1] + y1_ref[...]) * dis + b_ref[...]
        h = jnp.where(t >= 0, t, 0.01 * t)
        y2_ref[...] = jnp.dot(h, w_ref[...], preferred_element_type=jnp.float32) * dis

    return pl.pallas_call(
        body,
        grid=(N_NODES // BR,),
        in_specs=[
            pl.BlockSpec((2, BR, D), lambda i: (0, i, 0)),
            pl.BlockSpec((BR, D), lambda i: (i, 0)),
            pl.BlockSpec((2, BR, DEG_LANES), lambda i: (0, i, 0)),
            pl.BlockSpec((D, D), lambda i: (0, 0)),
            pl.BlockSpec((1, D), lambda i: (0, 0)),
        ],
        out_specs=pl.BlockSpec((BR, D), lambda i: (i, 0)),
        out_shape=jax.ShapeDtypeStruct((N_NODES, D), jnp.float32),
    )(agg, y1, deg, W2, b1.reshape(1, D))


def _tc_out(agg, y2, deg, b2):
    def body(agg_ref, y2_ref, deg_ref, b_ref, o_ref):
        dis = _dis_block(deg_ref)
        o_ref[...] = (agg_ref[0] + agg_ref[1] + y2_ref[...]) * dis + b_ref[...]

    return pl.pallas_call(
        body,
        grid=(N_NODES // BR,),
        in_specs=[
            pl.BlockSpec((2, BR, D), lambda i: (0, i, 0)),
            pl.BlockSpec((BR, D), lambda i: (i, 0)),
            pl.BlockSpec((2, BR, DEG_LANES), lambda i: (0, i, 0)),
            pl.BlockSpec((1, D), lambda i: (0, 0)),
        ],
        out_specs=pl.BlockSpec((BR, D), lambda i: (i, 0)),
        out_shape=jax.ShapeDtypeStruct((N_NODES, D), jnp.float32),
    )(agg, y2, deg, b2.reshape(1, D))


def kernel(x, edge_index, W1, b1, W2, b2):
    src3d = edge_index[0].astype(jnp.int32).reshape(NW, OUTER, IDX_BLK, CHUNK)
    dst3d = edge_index[1].astype(jnp.int32).reshape(NW, OUTER, IDX_BLK, CHUNK)

    deg = _sc_indeg(dst3d, jnp.zeros((N_PAD, DEG_LANES), jnp.float32))
    zrows = jnp.zeros((N_PAD, D), jnp.float32)
    y1 = _tc_y1(x, W1, deg)
    agg1 = _sc_agg(y1, src3d, dst3d, zrows)
    y2 = _tc_mid(agg1, y1, deg, W2, b1)
    agg2 = _sc_agg(y2, src3d, dst3d, zrows)
    return _tc_out(agg2, y2, deg, b2)


# speedup vs baseline: 23.2388x; 1.4949x over previous
import functools

import jax
import jax.numpy as jnp
from jax import lax
from jax.experimental import pallas as pl
from jax.experimental.pallas import tpu as pltpu
from jax.experimental.pallas import tpu_sc as plsc

N_NODES = 10000
D = 128
N_EDGES = 320000

NC = 2
NS = 16
NW = NC * NS
EPT = N_EDGES // NW
CHUNK = 80
NCHUNK = EPT // CHUNK
IDX_BLK = 5
OUTER = NCHUNK // IDX_BLK
N_PAD = 10240
RPT = N_PAD // NS
SLAB = 64
DEG_LANES = 128

_MESH = plsc.VectorSubcoreMesh(core_axis_name="c", subcore_axis_name="s")


def _zero_rows(buf, nrows, ncols):
    @pl.loop(0, nrows)
    def _(r):
        @pl.loop(0, ncols, step=16)
        def _(cc):
            buf[r, pl.ds(cc, 16)] = jnp.zeros((16,), jnp.float32)


def _sc_indeg(dst4d, zeros_rows):
    @functools.partial(
        pl.kernel,
        mesh=_MESH,
        out_type=jax.ShapeDtypeStruct((NC, N_PAD, DEG_LANES), jnp.float32),
        scratch_types=[
            pltpu.VMEM((IDX_BLK, CHUNK), jnp.int32),
            pltpu.VMEM((IDX_BLK, CHUNK), jnp.int32),
            pltpu.VMEM((CHUNK, DEG_LANES), jnp.float32),
            pltpu.VMEM_SHARED((N_PAD, DEG_LANES), jnp.float32),
            pltpu.SemaphoreType.DMA,
            pltpu.SemaphoreType.DMA,
        ],
    )
    def k(dst_hbm, z_hbm, out_hbm, idx0, idx1, ones_v, acc_s, sd0, sd1):
        c = lax.axis_index("c")
        s = lax.axis_index("s")
        wid = c * NS + s
        idxb, sd = [idx0, idx1], [sd0, sd1]

        @pl.loop(0, CHUNK)
        def _(r):
            ones_v[r, :] = jnp.ones((DEG_LANES,), jnp.float32)

        for k_ in range(NS):
            @pl.when(s == k_)
            def _(k_=k_):
                pltpu.sync_copy(z_hbm.at[pl.ds(k_ * RPT, RPT)],
                                acc_s.at[pl.ds(k_ * RPT, RPT)])

        plsc.subcore_barrier()

        pend = [[], []]
        for t in range(OUTER):
            bi = t & 1
            for h in pend[bi]:
                h.wait()
            pend[bi] = []
            pltpu.sync_copy(dst_hbm.at[wid, t], idxb[bi])
            for r in range(IDX_BLK):
                pend[bi].append(
                    pltpu.async_copy(ones_v, acc_s.at[idxb[bi].at[r]],
                                     sd[bi], add=True))
        for pl_ in pend:
            for h in pl_:
                h.wait()

        plsc.subcore_barrier()

        for k_ in range(NS):
            @pl.when(s == k_)
            def _(k_=k_):
                pltpu.sync_copy(acc_s.at[pl.ds(k_ * RPT, RPT)],
                                out_hbm.at[c, pl.ds(k_ * RPT, RPT)])

    return k(dst4d, zeros_rows)


def _sc_agg(y, src4d, dst4d, zrows):
    @functools.partial(
        pl.kernel,
        mesh=_MESH,
        out_type=jax.ShapeDtypeStruct((NC, N_PAD, D), jnp.float32),
        scratch_types=[
            pltpu.VMEM((IDX_BLK, CHUNK), jnp.int32),
            pltpu.VMEM((IDX_BLK, CHUNK), jnp.int32),
            pltpu.VMEM((IDX_BLK, CHUNK), jnp.int32),
            pltpu.VMEM((IDX_BLK, CHUNK), jnp.int32),
            pltpu.VMEM((CHUNK, D), jnp.float32),
            pltpu.VMEM((CHUNK, D), jnp.float32),
            pltpu.VMEM_SHARED((N_PAD, D), jnp.float32),
            pltpu.SemaphoreType.DMA,
            pltpu.SemaphoreType.DMA,
            pltpu.SemaphoreType.DMA,
            pltpu.SemaphoreType.DMA,
        ],
    )
    def k(y_hbm, src_hbm, dst_hbm, z_hbm, out_hbm,
          src0, src1, dst0, dst1, rows0, rows1, acc_s, sg0, sg1, ss0, ss1):
        c = lax.axis_index("c")
        s = lax.axis_index("s")
        wid = c * NS + s
        srcb, dstb = [src0, src1], [dst0, dst1]
        rows, sg, ss = [rows0, rows1], [sg0, sg1], [ss0, ss1]

        for k_ in range(NS):
            @pl.when(s == k_)
            def _(k_=k_):
                pltpu.sync_copy(z_hbm.at[pl.ds(k_ * RPT, RPT)],
                                acc_s.at[pl.ds(k_ * RPT, RPT)])

        plsc.subcore_barrier()

        gh = [None, None]
        sh = [None, None]
        gi = [None, None]
        for t in range(OUTER):
            bi = t & 1
            pltpu.sync_copy(src_hbm.at[wid, t], srcb[bi])
            pltpu.sync_copy(dst_hbm.at[wid, t], dstb[bi])
            for r in range(IDX_BLK):
                j = t * IDX_BLK + r
                p = j & 1
                q = 1 - p
                if sh[p] is not None:
                    sh[p].wait()
                    sh[p] = None
                gh[p] = pltpu.async_copy(y_hbm.at[srcb[bi].at[r]], rows[p],
                                         sg[p])
                gi[p] = (bi, r)
                if gh[q] is not None:
                    gh[q].wait()
                    gh[q] = None
                    pb, pr = gi[q]
                    sh[q] = pltpu.async_copy(rows[q],
                                             acc_s.at[dstb[pb].at[pr]],
                                             ss[q], add=True)
        for p in range(2):
            if gh[p] is not None:
                gh[p].wait()
                pb, pr = gi[p]
                sh[p] = pltpu.async_copy(rows[p], acc_s.at[dstb[pb].at[pr]],
                                         ss[p], add=True)
            if sh[p] is not None:
                sh[p].wait()

        plsc.subcore_barrier()

        for k_ in range(NS):
            @pl.when(s == k_)
            def _(k_=k_):
                pltpu.sync_copy(acc_s.at[pl.ds(k_ * RPT, RPT)],
                                out_hbm.at[c, pl.ds(k_ * RPT, RPT)])

    return k(y, src4d, dst4d, zrows)


BR = 400


def _dis_block(deg_ref):
    degsum = deg_ref[0] + deg_ref[1]
    return lax.rsqrt(1.0 + degsum[:, :1])


def _tc_y1(x, W1, deg):
    def body(x_ref, w_ref, deg_ref, y_ref):
        dis = _dis_block(deg_ref)
        xw = jnp.dot(x_ref[...], w_ref[...], preferred_element_type=jnp.float32)
        y_ref[...] = xw * dis

    return pl.pallas_call(
        body,
        grid=(N_NODES // BR,),
        in_specs=[
            pl.BlockSpec((BR, D), lambda i: (i, 0)),
            pl.BlockSpec((D, D), lambda i: (0, 0)),
            pl.BlockSpec((2, BR, DEG_LANES), lambda i: (0, i, 0)),
        ],
        out_specs=pl.BlockSpec((BR, D), lambda i: (i, 0)),
        out_shape=jax.ShapeDtypeStruct((N_NODES, D), jnp.float32),
    )(x, W1, deg)


def _tc_mid(agg, y1, deg, W2, b1):
    def body(agg_ref, y1_ref, deg_ref, w_ref, b_ref, y2_ref):
        dis = _dis_block(deg_ref)
        t = (agg_ref[0] + agg_ref[1] + y1_ref[...]) * dis + b_ref[...]
        h = jnp.where(t >= 0, t, 0.01 * t)
        y2_ref[...] = jnp.dot(h, w_ref[...], preferred_element_type=jnp.float32) * dis

    return pl.pallas_call(
        body,
        grid=(N_NODES // BR,),
        in_specs=[
            pl.BlockSpec((2, BR, D), lambda i: (0, i, 0)),
            pl.BlockSpec((BR, D), lambda i: (i, 0)),
            pl.BlockSpec((2, BR, DEG_LANES), lambda i: (0, i, 0)),
            pl.BlockSpec((D, D), lambda i: (0, 0)),
            pl.BlockSpec((1, D), lambda i: (0, 0)),
        ],
        out_specs=pl.BlockSpec((BR, D), lambda i: (i, 0)),
        out_shape=jax.ShapeDtypeStruct((N_NODES, D), jnp.float32),
    )(agg, y1, deg, W2, b1.reshape(1, D))


def _tc_out(agg, y2, deg, b2):
    def body(agg_ref, y2_ref, deg_ref, b_ref, o_ref):
        dis = _dis_block(deg_ref)
        o_ref[...] = (agg_ref[0] + agg_ref[1] + y2_ref[...]) * dis + b_ref[...]

    return pl.pallas_call(
        body,
        grid=(N_NODES // BR,),
        in_specs=[
            pl.BlockSpec((2, BR, D), lambda i: (0, i, 0)),
            pl.BlockSpec((BR, D), lambda i: (i, 0)),
            pl.BlockSpec((2, BR, DEG_LANES), lambda i: (0, i, 0)),
            pl.BlockSpec((1, D), lambda i: (0, 0)),
        ],
        out_specs=pl.BlockSpec((BR, D), lambda i: (i, 0)),
        out_shape=jax.ShapeDtypeStruct((N_NODES, D), jnp.float32),
    )(agg, y2, deg, b2.reshape(1, D))


def kernel(x, edge_index, W1, b1, W2, b2):
    src3d = edge_index[0].astype(jnp.int32).reshape(NW, OUTER, IDX_BLK, CHUNK)
    dst3d = edge_index[1].astype(jnp.int32).reshape(NW, OUTER, IDX_BLK, CHUNK)

    deg = _sc_indeg(dst3d, jnp.zeros((N_PAD, DEG_LANES), jnp.float32))
    zrows = jnp.zeros((N_PAD, D), jnp.float32)
    y1 = _tc_y1(x, W1, deg)
    agg1 = _sc_agg(y1, src3d, dst3d, zrows)
    y2 = _tc_mid(agg1, y1, deg, W2, b1)
    agg2 = _sc_agg(y2, src3d, dst3d, zrows)
    return _tc_out(agg2, y2, deg, b2)


# speedup vs baseline: 25.7124x; 1.1064x over previous
import functools

import jax
import jax.numpy as jnp
from jax import lax
from jax.experimental import pallas as pl
from jax.experimental.pallas import tpu as pltpu
from jax.experimental.pallas import tpu_sc as plsc

N_NODES = 10000
D = 128
N_EDGES = 320000

NC = 2
NS = 16
NW = NC * NS
EPT = N_EDGES // NW
CHUNK = 80
NCHUNK = EPT // CHUNK
IDX_BLK = 5
OUTER = NCHUNK // IDX_BLK
N_PAD = 10240
RPT = N_PAD // NS
SLAB = 64
DEG_LANES = 128

_MESH = plsc.VectorSubcoreMesh(core_axis_name="c", subcore_axis_name="s")


def _zero_rows(buf, nrows, ncols):
    @pl.loop(0, nrows)
    def _(r):
        @pl.loop(0, ncols, step=16)
        def _(cc):
            buf[r, pl.ds(cc, 16)] = jnp.zeros((16,), jnp.float32)


def _sc_indeg(dst4d, zeros_rows):
    @functools.partial(
        pl.kernel,
        mesh=_MESH,
        out_type=jax.ShapeDtypeStruct((NC, N_PAD, DEG_LANES), jnp.float32),
        scratch_types=[
            pltpu.VMEM((IDX_BLK, CHUNK), jnp.int32),
            pltpu.VMEM((IDX_BLK, CHUNK), jnp.int32),
            pltpu.VMEM((CHUNK, DEG_LANES), jnp.float32),
            pltpu.VMEM_SHARED((N_PAD, DEG_LANES), jnp.float32),
            pltpu.SemaphoreType.DMA,
            pltpu.SemaphoreType.DMA,
        ],
    )
    def k(dst_hbm, z_hbm, out_hbm, idx0, idx1, ones_v, acc_s, sd0, sd1):
        c = lax.axis_index("c")
        s = lax.axis_index("s")
        wid = c * NS + s
        idxb, sd = [idx0, idx1], [sd0, sd1]

        @pl.loop(0, CHUNK)
        def _(r):
            ones_v[r, :] = jnp.ones((DEG_LANES,), jnp.float32)

        for k_ in range(NS):
            @pl.when(s == k_)
            def _(k_=k_):
                pltpu.sync_copy(z_hbm.at[pl.ds(k_ * RPT, RPT)],
                                acc_s.at[pl.ds(k_ * RPT, RPT)])

        plsc.subcore_barrier()

        pend = [[], []]
        for t in range(OUTER):
            bi = t & 1
            for h in pend[bi]:
                h.wait()
            pend[bi] = []
            pltpu.sync_copy(dst_hbm.at[wid, t], idxb[bi])
            for r in range(IDX_BLK):
                pend[bi].append(
                    pltpu.async_copy(ones_v, acc_s.at[idxb[bi].at[r]],
                                     sd[bi], add=True))
        for pl_ in pend:
            for h in pl_:
                h.wait()

        plsc.subcore_barrier()

        for k_ in range(NS):
            @pl.when(s == k_)
            def _(k_=k_):
                pltpu.sync_copy(acc_s.at[pl.ds(k_ * RPT, RPT)],
                                out_hbm.at[c, pl.ds(k_ * RPT, RPT)])

    return k(dst4d, zeros_rows)


def _sc_agg(y, src4d, dst4d, zrows):
    @functools.partial(
        pl.kernel,
        mesh=_MESH,
        out_type=jax.ShapeDtypeStruct((NC, N_PAD, D), jnp.float32),
        scratch_types=[
            pltpu.VMEM((IDX_BLK, CHUNK), jnp.int32),
            pltpu.VMEM((IDX_BLK, CHUNK), jnp.int32),
            pltpu.VMEM((IDX_BLK, CHUNK), jnp.int32),
            pltpu.VMEM((IDX_BLK, CHUNK), jnp.int32),
            pltpu.VMEM((CHUNK, D), jnp.float32),
            pltpu.VMEM((CHUNK, D), jnp.float32),
            pltpu.VMEM((CHUNK, D), jnp.float32),
            pltpu.VMEM_SHARED((N_PAD, D), jnp.float32),
            pltpu.SemaphoreType.DMA,
            pltpu.SemaphoreType.DMA,
            pltpu.SemaphoreType.DMA,
            pltpu.SemaphoreType.DMA,
            pltpu.SemaphoreType.DMA,
            pltpu.SemaphoreType.DMA,
        ],
    )
    def k(y_hbm, src_hbm, dst_hbm, z_hbm, out_hbm,
          src0, src1, dst0, dst1, rows0, rows1, rows2, acc_s,
          sg0, sg1, sg2, ss0, ss1, ss2):
        c = lax.axis_index("c")
        s = lax.axis_index("s")
        wid = c * NS + s
        srcb, dstb = [src0, src1], [dst0, dst1]
        rows = [rows0, rows1, rows2]
        sg, ss = [sg0, sg1, sg2], [ss0, ss1, ss2]

        for k_ in range(NS):
            @pl.when(s == k_)
            def _(k_=k_):
                pltpu.sync_copy(z_hbm.at[pl.ds(k_ * RPT, RPT)],
                                acc_s.at[pl.ds(k_ * RPT, RPT)])

        plsc.subcore_barrier()

        NBUF = 3
        gh = [None] * NBUF
        sh = [None] * NBUF
        gi = [None] * NBUF

        def issue_scatter(jq):
            q = jq % NBUF
            gh[q].wait()
            gh[q] = None
            pb, pr = gi[q]
            sh[q] = pltpu.async_copy(rows[q], acc_s.at[dstb[pb].at[pr]],
                                     ss[q], add=True)

        for t in range(OUTER):
            bi = t & 1
            pltpu.sync_copy(src_hbm.at[wid, t], srcb[bi])
            pltpu.sync_copy(dst_hbm.at[wid, t], dstb[bi])
            for r in range(IDX_BLK):
                j = t * IDX_BLK + r
                p = j % NBUF
                if sh[p] is not None:
                    sh[p].wait()
                    sh[p] = None
                gh[p] = pltpu.async_copy(y_hbm.at[srcb[bi].at[r]], rows[p],
                                         sg[p])
                gi[p] = (bi, r)
                if j >= NBUF - 1:
                    issue_scatter(j - (NBUF - 1))
        for jq in range(NCHUNK - NBUF + 1, NCHUNK):
            issue_scatter(jq)
        for p in range(NBUF):
            if sh[p] is not None:
                sh[p].wait()

        plsc.subcore_barrier()

        for k_ in range(NS):
            @pl.when(s == k_)
            def _(k_=k_):
                pltpu.sync_copy(acc_s.at[pl.ds(k_ * RPT, RPT)],
                                out_hbm.at[c, pl.ds(k_ * RPT, RPT)])

    return k(y, src4d, dst4d, zrows)


BR = 400


def _dis_block(deg_ref):
    degsum = deg_ref[0] + deg_ref[1]
    return lax.rsqrt(1.0 + degsum[:, :1])


def _tc_y1(x, W1, deg):
    def body(x_ref, w_ref, deg_ref, y_ref):
        dis = _dis_block(deg_ref)
        xw = jnp.dot(x_ref[...], w_ref[...], preferred_element_type=jnp.float32)
        y_ref[...] = xw * dis

    return pl.pallas_call(
        body,
        grid=(N_NODES // BR,),
        in_specs=[
            pl.BlockSpec((BR, D), lambda i: (i, 0)),
            pl.BlockSpec((D, D), lambda i: (0, 0)),
            pl.BlockSpec((2, BR, DEG_LANES), lambda i: (0, i, 0)),
        ],
        out_specs=pl.BlockSpec((BR, D), lambda i: (i, 0)),
        out_shape=jax.ShapeDtypeStruct((N_NODES, D), jnp.float32),
    )(x, W1, deg)


def _tc_mid(agg, y1, deg, W2, b1):
    def body(agg_ref, y1_ref, deg_ref, w_ref, b_ref, y2_ref):
        dis = _dis_block(deg_ref)
        t = (agg_ref[0] + agg_ref[1] + y1_ref[...]) * dis + b_ref[...]
        h = jnp.where(t >= 0, t, 0.01 * t)
        y2_ref[...] = jnp.dot(h, w_ref[...], preferred_element_type=jnp.float32) * dis

    return pl.pallas_call(
        body,
        grid=(N_NODES // BR,),
        in_specs=[
            pl.BlockSpec((2, BR, D), lambda i: (0, i, 0)),
            pl.BlockSpec((BR, D), lambda i: (i, 0)),
            pl.BlockSpec((2, BR, DEG_LANES), lambda i: (0, i, 0)),
            pl.BlockSpec((D, D), lambda i: (0, 0)),
            pl.BlockSpec((1, D), lambda i: (0, 0)),
        ],
        out_specs=pl.BlockSpec((BR, D), lambda i: (i, 0)),
        out_shape=jax.ShapeDtypeStruct((N_NODES, D), jnp.float32),
    )(agg, y1, deg, W2, b1.reshape(1, D))


def _tc_out(agg, y2, deg, b2):
    def body(agg_ref, y2_ref, deg_ref, b_ref, o_ref):
        dis = _dis_block(deg_ref)
        o_ref[...] = (agg_ref[0] + agg_ref[1] + y2_ref[...]) * dis + b_ref[...]

    return pl.pallas_call(
        body,
        grid=(N_NODES // BR,),
        in_specs=[
            pl.BlockSpec((2, BR, D), lambda i: (0, i, 0)),
            pl.BlockSpec((BR, D), lambda i: (i, 0)),
            pl.BlockSpec((2, BR, DEG_LANES), lambda i: (0, i, 0)),
            pl.BlockSpec((1, D), lambda i: (0, 0)),
        ],
        out_specs=pl.BlockSpec((BR, D), lambda i: (i, 0)),
        out_shape=jax.ShapeDtypeStruct((N_NODES, D), jnp.float32),
    )(agg, y2, deg, b2.reshape(1, D))


def kernel(x, edge_index, W1, b1, W2, b2):
    src3d = edge_index[0].astype(jnp.int32).reshape(NW, OUTER, IDX_BLK, CHUNK)
    dst3d = edge_index[1].astype(jnp.int32).reshape(NW, OUTER, IDX_BLK, CHUNK)

    deg = _sc_indeg(dst3d, jnp.zeros((N_PAD, DEG_LANES), jnp.float32))
    zrows = jnp.zeros((N_PAD, D), jnp.float32)
    y1 = _tc_y1(x, W1, deg)
    agg1 = _sc_agg(y1, src3d, dst3d, zrows)
    y2 = _tc_mid(agg1, y1, deg, W2, b1)
    agg2 = _sc_agg(y2, src3d, dst3d, zrows)
    return _tc_out(agg2, y2, deg, b2)


# speedup vs baseline: 25.7579x; 1.0018x over previous
import functools

import jax
import jax.numpy as jnp
from jax import lax
from jax.experimental import pallas as pl
from jax.experimental.pallas import tpu as pltpu
from jax.experimental.pallas import tpu_sc as plsc

N_NODES = 10000
D = 128
N_EDGES = 320000

NC = 2
NS = 16
NW = NC * NS
EPT = N_EDGES // NW
CHUNK = 80
NCHUNK = EPT // CHUNK
IDX_BLK = 5
OUTER = NCHUNK // IDX_BLK
N_PAD = 10240
RPT = N_PAD // NS
SLAB = 64
DEG_LANES = 128

_MESH = plsc.VectorSubcoreMesh(core_axis_name="c", subcore_axis_name="s")


def _zero_rows(buf, nrows, ncols):
    @pl.loop(0, nrows)
    def _(r):
        @pl.loop(0, ncols, step=16)
        def _(cc):
            buf[r, pl.ds(cc, 16)] = jnp.zeros((16,), jnp.float32)


def _sc_indeg(dst4d, zeros_rows):
    @functools.partial(
        pl.kernel,
        mesh=_MESH,
        out_type=jax.ShapeDtypeStruct((NC, N_PAD, DEG_LANES), jnp.float32),
        scratch_types=[
            pltpu.VMEM((IDX_BLK, CHUNK), jnp.int32),
            pltpu.VMEM((IDX_BLK, CHUNK), jnp.int32),
            pltpu.VMEM((CHUNK, DEG_LANES), jnp.float32),
            pltpu.VMEM_SHARED((N_PAD, DEG_LANES), jnp.float32),
            pltpu.SemaphoreType.DMA,
            pltpu.SemaphoreType.DMA,
        ],
    )
    def k(dst_hbm, z_hbm, out_hbm, idx0, idx1, ones_v, acc_s, sd0, sd1):
        c = lax.axis_index("c")
        s = lax.axis_index("s")
        wid = c * NS + s
        idxb, sd = [idx0, idx1], [sd0, sd1]

        @pl.loop(0, CHUNK)
        def _(r):
            ones_v[r, :] = jnp.ones((DEG_LANES,), jnp.float32)

        for k_ in range(NS):
            @pl.when(s == k_)
            def _(k_=k_):
                pltpu.sync_copy(z_hbm.at[pl.ds(k_ * RPT, RPT)],
                                acc_s.at[pl.ds(k_ * RPT, RPT)])

        plsc.subcore_barrier()

        pend = [[], []]
        for t in range(OUTER):
            bi = t & 1
            for h in pend[bi]:
                h.wait()
            pend[bi] = []
            pltpu.sync_copy(dst_hbm.at[wid, t], idxb[bi])
            for r in range(IDX_BLK):
                pend[bi].append(
                    pltpu.async_copy(ones_v, acc_s.at[idxb[bi].at[r]],
                                     sd[bi], add=True))
        for pl_ in pend:
            for h in pl_:
                h.wait()

        plsc.subcore_barrier()

        for k_ in range(NS):
            @pl.when(s == k_)
            def _(k_=k_):
                pltpu.sync_copy(acc_s.at[pl.ds(k_ * RPT, RPT)],
                                out_hbm.at[c, pl.ds(k_ * RPT, RPT)])

    return k(dst4d, zeros_rows)


def _sc_agg(y, src4d, dst4d, zrows):
    @functools.partial(
        pl.kernel,
        mesh=_MESH,
        out_type=jax.ShapeDtypeStruct((NC, N_PAD, D), jnp.float32),
        scratch_types=[
            pltpu.VMEM((IDX_BLK, CHUNK), jnp.int32),
            pltpu.VMEM((IDX_BLK, CHUNK), jnp.int32),
            pltpu.VMEM((IDX_BLK, CHUNK), jnp.int32),
            pltpu.VMEM((IDX_BLK, CHUNK), jnp.int32),
            pltpu.VMEM((CHUNK, D), jnp.float32),
            pltpu.VMEM((CHUNK, D), jnp.float32),
            pltpu.VMEM((CHUNK, D), jnp.float32),
            pltpu.VMEM_SHARED((N_PAD, D), jnp.float32),
            pltpu.SemaphoreType.DMA,
            pltpu.SemaphoreType.DMA,
            pltpu.SemaphoreType.DMA,
            pltpu.SemaphoreType.DMA,
            pltpu.SemaphoreType.DMA,
            pltpu.SemaphoreType.DMA,
        ],
    )
    def k(y_hbm, src_hbm, dst_hbm, z_hbm, out_hbm,
          src0, src1, dst0, dst1, rows0, rows1, rows2, acc_s,
          sg0, sg1, sg2, ss0, ss1, ss2):
        c = lax.axis_index("c")
        s = lax.axis_index("s")
        wid = c * NS + s
        srcb, dstb = [src0, src1], [dst0, dst1]
        rows = [rows0, rows1, rows2]
        sg, ss = [sg0, sg1, sg2], [ss0, ss1, ss2]

        for k_ in range(NS):
            @pl.when(s == k_)
            def _(k_=k_):
                pltpu.sync_copy(z_hbm.at[pl.ds(k_ * RPT, RPT)],
                                acc_s.at[pl.ds(k_ * RPT, RPT)])

        plsc.subcore_barrier()

        NBUF = 3
        gh = [None] * NBUF
        sh = [None] * NBUF
        gi = [None] * NBUF

        def issue_scatter(jq):
            q = jq % NBUF
            gh[q].wait()
            gh[q] = None
            pb, pr = gi[q]
            sh[q] = pltpu.async_copy(rows[q], acc_s.at[dstb[pb].at[pr]],
                                     ss[q], add=True)

        for t in range(OUTER):
            bi = t & 1
            pltpu.sync_copy(src_hbm.at[wid, t], srcb[bi])
            pltpu.sync_copy(dst_hbm.at[wid, t], dstb[bi])
            for r in range(IDX_BLK):
                j = t * IDX_BLK + r
                p = j % NBUF
                if sh[p] is not None:
                    sh[p].wait()
                    sh[p] = None
                gh[p] = pltpu.async_copy(y_hbm.at[srcb[bi].at[r]], rows[p],
                                         sg[p])
                gi[p] = (bi, r)
                if j >= NBUF - 1:
                    issue_scatter(j - (NBUF - 1))
        for jq in range(NCHUNK - NBUF + 1, NCHUNK):
            issue_scatter(jq)
        for p in range(NBUF):
            if sh[p] is not None:
                sh[p].wait()

        plsc.subcore_barrier()

        for k_ in range(NS):
            @pl.when(s == k_)
            def _(k_=k_):
                pltpu.sync_copy(acc_s.at[pl.ds(k_ * RPT, RPT)],
                                out_hbm.at[c, pl.ds(k_ * RPT, RPT)])

    return k(y, src4d, dst4d, zrows)


BR = 400


def _dis_block(deg_ref):
    degsum = deg_ref[0] + deg_ref[1]
    return lax.rsqrt(1.0 + degsum[:, :1])


def _tc_xw(x, W1):
    def body(x_ref, w_ref, o_ref):
        o_ref[...] = jnp.dot(x_ref[...], w_ref[...],
                             preferred_element_type=jnp.float32)

    return pl.pallas_call(
        body,
        grid=(N_NODES // BR,),
        in_specs=[
            pl.BlockSpec((BR, D), lambda i: (i, 0)),
            pl.BlockSpec((D, D), lambda i: (0, 0)),
        ],
        out_specs=pl.BlockSpec((BR, D), lambda i: (i, 0)),
        out_shape=jax.ShapeDtypeStruct((N_NODES, D), jnp.float32),
    )(x, W1)


def _tc_scale(xw, deg):
    def body(xw_ref, deg_ref, y_ref):
        y_ref[...] = xw_ref[...] * _dis_block(deg_ref)

    return pl.pallas_call(
        body,
        grid=(N_NODES // BR,),
        in_specs=[
            pl.BlockSpec((BR, D), lambda i: (i, 0)),
            pl.BlockSpec((2, BR, DEG_LANES), lambda i: (0, i, 0)),
        ],
        out_specs=pl.BlockSpec((BR, D), lambda i: (i, 0)),
        out_shape=jax.ShapeDtypeStruct((N_NODES, D), jnp.float32),
    )(xw, deg)


def _tc_mid(agg, y1, deg, W2, b1):
    def body(agg_ref, y1_ref, deg_ref, w_ref, b_ref, y2_ref):
        dis = _dis_block(deg_ref)
        t = (agg_ref[0] + agg_ref[1] + y1_ref[...]) * dis + b_ref[...]
        h = jnp.where(t >= 0, t, 0.01 * t)
        y2_ref[...] = jnp.dot(h, w_ref[...], preferred_element_type=jnp.float32) * dis

    return pl.pallas_call(
        body,
        grid=(N_NODES // BR,),
        in_specs=[
            pl.BlockSpec((2, BR, D), lambda i: (0, i, 0)),
            pl.BlockSpec((BR, D), lambda i: (i, 0)),
            pl.BlockSpec((2, BR, DEG_LANES), lambda i: (0, i, 0)),
            pl.BlockSpec((D, D), lambda i: (0, 0)),
            pl.BlockSpec((1, D), lambda i: (0, 0)),
        ],
        out_specs=pl.BlockSpec((BR, D), lambda i: (i, 0)),
        out_shape=jax.ShapeDtypeStruct((N_NODES, D), jnp.float32),
    )(agg, y1, deg, W2, b1.reshape(1, D))


def _tc_out(agg, y2, deg, b2):
    def body(agg_ref, y2_ref, deg_ref, b_ref, o_ref):
        dis = _dis_block(deg_ref)
        o_ref[...] = (agg_ref[0] + agg_ref[1] + y2_ref[...]) * dis + b_ref[...]

    return pl.pallas_call(
        body,
        grid=(N_NODES // BR,),
        in_specs=[
            pl.BlockSpec((2, BR, D), lambda i: (0, i, 0)),
            pl.BlockSpec((BR, D), lambda i: (i, 0)),
            pl.BlockSpec((2, BR, DEG_LANES), lambda i: (0, i, 0)),
            pl.BlockSpec((1, D), lambda i: (0, 0)),
        ],
        out_specs=pl.BlockSpec((BR, D), lambda i: (i, 0)),
        out_shape=jax.ShapeDtypeStruct((N_NODES, D), jnp.float32),
    )(agg, y2, deg, b2.reshape(1, D))


def kernel(x, edge_index, W1, b1, W2, b2):
    src3d = edge_index[0].astype(jnp.int32).reshape(NW, OUTER, IDX_BLK, CHUNK)
    dst3d = edge_index[1].astype(jnp.int32).reshape(NW, OUTER, IDX_BLK, CHUNK)

    xw = _tc_xw(x, W1)
    deg = _sc_indeg(dst3d, jnp.zeros((N_PAD, DEG_LANES), jnp.float32))
    zrows = jnp.zeros((N_PAD, D), jnp.float32)
    y1 = _tc_scale(xw, deg)
    agg1 = _sc_agg(y1, src3d, dst3d, zrows)
    y2 = _tc_mid(agg1, y1, deg, W2, b1)
    agg2 = _sc_agg(y2, src3d, dst3d, zrows)
    return _tc_out(agg2, y2, deg, b2)


# speedup vs baseline: 26.1284x; 1.0144x over previous
import functools

import jax
import jax.numpy as jnp
from jax import lax
from jax.experimental import pallas as pl
from jax.experimental.pallas import tpu as pltpu
from jax.experimental.pallas import tpu_sc as plsc

N_NODES = 10000
D = 128
N_EDGES = 320000

NC = 2
NS = 16
NW = NC * NS
EPT = N_EDGES // NW
CHUNK = 80
NCHUNK = EPT // CHUNK
IDX_BLK = 5
OUTER = NCHUNK // IDX_BLK
N_PAD = 10240
RPT = N_PAD // NS
SLAB = 64
DEG_LANES = 128

_MESH = plsc.VectorSubcoreMesh(core_axis_name="c", subcore_axis_name="s")


def _zero_rows(buf, nrows, ncols):
    @pl.loop(0, nrows)
    def _(r):
        @pl.loop(0, ncols, step=16)
        def _(cc):
            buf[r, pl.ds(cc, 16)] = jnp.zeros((16,), jnp.float32)


def _sc_indeg(dst4d, zeros_rows):
    @functools.partial(
        pl.kernel,
        mesh=_MESH,
        out_type=jax.ShapeDtypeStruct((NC, N_PAD, DEG_LANES), jnp.float32),
        scratch_types=[
            pltpu.VMEM((IDX_BLK, CHUNK), jnp.int32),
            pltpu.VMEM((IDX_BLK, CHUNK), jnp.int32),
            pltpu.VMEM((CHUNK, DEG_LANES), jnp.float32),
            pltpu.VMEM_SHARED((N_PAD, DEG_LANES), jnp.float32),
            pltpu.SemaphoreType.DMA,
            pltpu.SemaphoreType.DMA,
        ],
    )
    def k(dst_hbm, z_hbm, out_hbm, idx0, idx1, ones_v, acc_s, sd0, sd1):
        c = lax.axis_index("c")
        s = lax.axis_index("s")
        wid = c * NS + s
        idxb, sd = [idx0, idx1], [sd0, sd1]

        @pl.loop(0, CHUNK)
        def _(r):
            ones_v[r, :] = jnp.ones((DEG_LANES,), jnp.float32)

        for k_ in range(NS):
            @pl.when(s == k_)
            def _(k_=k_):
                pltpu.sync_copy(z_hbm.at[pl.ds(k_ * RPT, RPT)],
                                acc_s.at[pl.ds(k_ * RPT, RPT)])

        plsc.subcore_barrier()

        pend = [[], []]
        for t in range(OUTER):
            bi = t & 1
            for h in pend[bi]:
                h.wait()
            pend[bi] = []
            pltpu.sync_copy(dst_hbm.at[wid, t], idxb[bi])
            for r in range(IDX_BLK):
                pend[bi].append(
                    pltpu.async_copy(ones_v, acc_s.at[idxb[bi].at[r]],
                                     sd[bi], add=True))
        for pl_ in pend:
            for h in pl_:
                h.wait()

        plsc.subcore_barrier()

        for k_ in range(NS):
            @pl.when(s == k_)
            def _(k_=k_):
                pltpu.sync_copy(acc_s.at[pl.ds(k_ * RPT, RPT)],
                                out_hbm.at[c, pl.ds(k_ * RPT, RPT)])

    return k(dst4d, zeros_rows)


def _sc_agg(y, src4d, dst4d, zrows):
    @functools.partial(
        pl.kernel,
        mesh=_MESH,
        out_type=jax.ShapeDtypeStruct((NC, N_PAD, D), jnp.float32),
        scratch_types=[
            pltpu.VMEM((IDX_BLK, CHUNK), jnp.int32),
            pltpu.VMEM((IDX_BLK, CHUNK), jnp.int32),
            pltpu.VMEM((IDX_BLK, CHUNK), jnp.int32),
            pltpu.VMEM((IDX_BLK, CHUNK), jnp.int32),
            pltpu.VMEM((CHUNK, D), jnp.float32),
            pltpu.VMEM((CHUNK, D), jnp.float32),
            pltpu.VMEM((CHUNK, D), jnp.float32),
            pltpu.VMEM((CHUNK, D), jnp.float32),
            pltpu.VMEM_SHARED((N_PAD, D), jnp.float32),
            pltpu.SemaphoreType.DMA,
            pltpu.SemaphoreType.DMA,
            pltpu.SemaphoreType.DMA,
            pltpu.SemaphoreType.DMA,
            pltpu.SemaphoreType.DMA,
            pltpu.SemaphoreType.DMA,
            pltpu.SemaphoreType.DMA,
            pltpu.SemaphoreType.DMA,
        ],
    )
    def k(y_hbm, src_hbm, dst_hbm, z_hbm, out_hbm,
          src0, src1, dst0, dst1, rows0, rows1, rows2, rows3, acc_s,
          sg0, sg1, sg2, sg3, ss0, ss1, ss2, ss3):
        c = lax.axis_index("c")
        s = lax.axis_index("s")
        wid = c * NS + s
        srcb, dstb = [src0, src1], [dst0, dst1]
        rows = [rows0, rows1, rows2, rows3]
        sg, ss = [sg0, sg1, sg2, sg3], [ss0, ss1, ss2, ss3]

        for k_ in range(NS):
            @pl.when(s == k_)
            def _(k_=k_):
                pltpu.sync_copy(z_hbm.at[pl.ds(k_ * RPT, RPT)],
                                acc_s.at[pl.ds(k_ * RPT, RPT)])

        plsc.subcore_barrier()

        NBUF = 4
        gh = [None] * NBUF
        sh = [None] * NBUF
        gi = [None] * NBUF

        def issue_scatter(jq):
            q = jq % NBUF
            gh[q].wait()
            gh[q] = None
            pb, pr = gi[q]
            sh[q] = pltpu.async_copy(rows[q], acc_s.at[dstb[pb].at[pr]],
                                     ss[q], add=True)

        for t in range(OUTER):
            bi = t & 1
            pltpu.sync_copy(src_hbm.at[wid, t], srcb[bi])
            pltpu.sync_copy(dst_hbm.at[wid, t], dstb[bi])
            for r in range(IDX_BLK):
                j = t * IDX_BLK + r
                p = j % NBUF
                if sh[p] is not None:
                    sh[p].wait()
                    sh[p] = None
                gh[p] = pltpu.async_copy(y_hbm.at[srcb[bi].at[r]], rows[p],
                                         sg[p])
                gi[p] = (bi, r)
                if j >= NBUF - 1:
                    issue_scatter(j - (NBUF - 1))
        for jq in range(NCHUNK - NBUF + 1, NCHUNK):
            issue_scatter(jq)
        for p in range(NBUF):
            if sh[p] is not None:
                sh[p].wait()

        plsc.subcore_barrier()

        for k_ in range(NS):
            @pl.when(s == k_)
            def _(k_=k_):
                pltpu.sync_copy(acc_s.at[pl.ds(k_ * RPT, RPT)],
                                out_hbm.at[c, pl.ds(k_ * RPT, RPT)])

    return k(y, src4d, dst4d, zrows)


BR = 400


def _dis_block(deg_ref):
    degsum = deg_ref[0] + deg_ref[1]
    return lax.rsqrt(1.0 + degsum[:, :1])


def _tc_xw(x, W1):
    def body(x_ref, w_ref, o_ref):
        o_ref[...] = jnp.dot(x_ref[...], w_ref[...],
                             preferred_element_type=jnp.float32)

    return pl.pallas_call(
        body,
        grid=(N_NODES // BR,),
        in_specs=[
            pl.BlockSpec((BR, D), lambda i: (i, 0)),
            pl.BlockSpec((D, D), lambda i: (0, 0)),
        ],
        out_specs=pl.BlockSpec((BR, D), lambda i: (i, 0)),
        out_shape=jax.ShapeDtypeStruct((N_NODES, D), jnp.float32),
    )(x, W1)


def _tc_scale(xw, deg):
    def body(xw_ref, deg_ref, y_ref):
        y_ref[...] = xw_ref[...] * _dis_block(deg_ref)

    return pl.pallas_call(
        body,
        grid=(N_NODES // BR,),
        in_specs=[
            pl.BlockSpec((BR, D), lambda i: (i, 0)),
            pl.BlockSpec((2, BR, DEG_LANES), lambda i: (0, i, 0)),
        ],
        out_specs=pl.BlockSpec((BR, D), lambda i: (i, 0)),
        out_shape=jax.ShapeDtypeStruct((N_NODES, D), jnp.float32),
    )(xw, deg)


def _tc_mid(agg, y1, deg, W2, b1):
    def body(agg_ref, y1_ref, deg_ref, w_ref, b_ref, y2_ref):
        dis = _dis_block(deg_ref)
        t = (agg_ref[0] + agg_ref[1] + y1_ref[...]) * dis + b_ref[...]
        h = jnp.where(t >= 0, t, 0.01 * t)
        y2_ref[...] = jnp.dot(h, w_ref[...], preferred_element_type=jnp.float32) * dis

    return pl.pallas_call(
        body,
        grid=(N_NODES // BR,),
        in_specs=[
            pl.BlockSpec((2, BR, D), lambda i: (0, i, 0)),
            pl.BlockSpec((BR, D), lambda i: (i, 0)),
            pl.BlockSpec((2, BR, DEG_LANES), lambda i: (0, i, 0)),
            pl.BlockSpec((D, D), lambda i: (0, 0)),
            pl.BlockSpec((1, D), lambda i: (0, 0)),
        ],
        out_specs=pl.BlockSpec((BR, D), lambda i: (i, 0)),
        out_shape=jax.ShapeDtypeStruct((N_NODES, D), jnp.float32),
    )(agg, y1, deg, W2, b1.reshape(1, D))


def _tc_out(agg, y2, deg, b2):
    def body(agg_ref, y2_ref, deg_ref, b_ref, o_ref):
        dis = _dis_block(deg_ref)
        o_ref[...] = (agg_ref[0] + agg_ref[1] + y2_ref[...]) * dis + b_ref[...]

    return pl.pallas_call(
        body,
        grid=(N_NODES // BR,),
        in_specs=[
            pl.BlockSpec((2, BR, D), lambda i: (0, i, 0)),
            pl.BlockSpec((BR, D), lambda i: (i, 0)),
            pl.BlockSpec((2, BR, DEG_LANES), lambda i: (0, i, 0)),
            pl.BlockSpec((1, D), lambda i: (0, 0)),
        ],
        out_specs=pl.BlockSpec((BR, D), lambda i: (i, 0)),
        out_shape=jax.ShapeDtypeStruct((N_NODES, D), jnp.float32),
    )(agg, y2, deg, b2.reshape(1, D))


def kernel(x, edge_index, W1, b1, W2, b2):
    src3d = edge_index[0].astype(jnp.int32).reshape(NW, OUTER, IDX_BLK, CHUNK)
    dst3d = edge_index[1].astype(jnp.int32).reshape(NW, OUTER, IDX_BLK, CHUNK)

    xw = _tc_xw(x, W1)
    deg = _sc_indeg(dst3d, jnp.zeros((N_PAD, DEG_LANES), jnp.float32))
    zrows = jnp.zeros((N_PAD, D), jnp.float32)
    y1 = _tc_scale(xw, deg)
    agg1 = _sc_agg(y1, src3d, dst3d, zrows)
    y2 = _tc_mid(agg1, y1, deg, W2, b1)
    agg2 = _sc_agg(y2, src3d, dst3d, zrows)
    return _tc_out(agg2, y2, deg, b2)


# speedup vs baseline: 28.1593x; 1.0777x over previous
import functools

import jax
import jax.numpy as jnp
from jax import lax
from jax.experimental import pallas as pl
from jax.experimental.pallas import tpu as pltpu
from jax.experimental.pallas import tpu_sc as plsc

N_NODES = 10000
D = 128
N_EDGES = 320000

NC = 2
NS = 16
NW = NC * NS
EPT = N_EDGES // NW
CHUNK = 80
NCHUNK = EPT // CHUNK
IDX_BLK = 5
OUTER = NCHUNK // IDX_BLK
N_PAD = 10240
RPT = N_PAD // NS
SLAB = 64
DEG_LANES = 128

_MESH = plsc.VectorSubcoreMesh(core_axis_name="c", subcore_axis_name="s")


def _zero_rows(buf, nrows, ncols):
    @pl.loop(0, nrows)
    def _(r):
        @pl.loop(0, ncols, step=16)
        def _(cc):
            buf[r, pl.ds(cc, 16)] = jnp.zeros((16,), jnp.float32)


def _sc_indeg(dst4d, zeros_rows):
    @functools.partial(
        pl.kernel,
        mesh=_MESH,
        out_type=jax.ShapeDtypeStruct((NC, N_PAD, DEG_LANES), jnp.float32),
        scratch_types=[
            pltpu.VMEM((IDX_BLK, CHUNK), jnp.int32),
            pltpu.VMEM((IDX_BLK, CHUNK), jnp.int32),
            pltpu.VMEM((IDX_BLK, CHUNK), jnp.int32),
            pltpu.VMEM((CHUNK, DEG_LANES), jnp.float32),
            pltpu.VMEM_SHARED((N_PAD, DEG_LANES), jnp.float32),
            pltpu.SemaphoreType.DMA,
            pltpu.SemaphoreType.DMA,
            pltpu.SemaphoreType.DMA,
            pltpu.SemaphoreType.DMA,
            pltpu.SemaphoreType.DMA,
            pltpu.SemaphoreType.DMA,
        ],
    )
    def k(dst_hbm, z_hbm, out_hbm, idx0, idx1, idx2, ones_v, acc_s,
          sd0, sd1, sd2, si0, si1, si2):
        c = lax.axis_index("c")
        s = lax.axis_index("s")
        wid = c * NS + s
        idxb, sd = [idx0, idx1, idx2], [sd0, sd1, sd2]
        si = [si0, si1, si2]

        @pl.loop(0, CHUNK)
        def _(r):
            ones_v[r, :] = jnp.ones((DEG_LANES,), jnp.float32)

        for k_ in range(NS):
            @pl.when(s == k_)
            def _(k_=k_):
                pltpu.sync_copy(z_hbm.at[pl.ds(k_ * RPT, RPT)],
                                acc_s.at[pl.ds(k_ * RPT, RPT)])

        plsc.subcore_barrier()

        pend = [[], [], []]
        ih = [None, None, None]
        ih[0] = pltpu.async_copy(dst_hbm.at[wid, 0], idxb[0], si[0])
        for t in range(OUTER):
            bi = t % 3
            ih[bi].wait()
            if t + 1 < OUTER:
                nb = (t + 1) % 3
                for h in pend[nb]:
                    h.wait()
                pend[nb] = []
                ih[nb] = pltpu.async_copy(dst_hbm.at[wid, t + 1], idxb[nb],
                                          si[nb])
            for h in pend[bi]:
                h.wait()
            pend[bi] = []
            for r in range(IDX_BLK):
                pend[bi].append(
                    pltpu.async_copy(ones_v, acc_s.at[idxb[bi].at[r]],
                                     sd[bi], add=True))
        for pl_ in pend:
            for h in pl_:
                h.wait()

        plsc.subcore_barrier()

        for k_ in range(NS):
            @pl.when(s == k_)
            def _(k_=k_):
                pltpu.sync_copy(acc_s.at[pl.ds(k_ * RPT, RPT)],
                                out_hbm.at[c, pl.ds(k_ * RPT, RPT)])

    return k(dst4d, zeros_rows)


def _sc_agg(y, src4d, dst4d, zrows):
    @functools.partial(
        pl.kernel,
        mesh=_MESH,
        out_type=jax.ShapeDtypeStruct((NC, N_PAD, D), jnp.float32),
        scratch_types=[
            pltpu.VMEM((IDX_BLK, CHUNK), jnp.int32),
            pltpu.VMEM((IDX_BLK, CHUNK), jnp.int32),
            pltpu.VMEM((IDX_BLK, CHUNK), jnp.int32),
            pltpu.VMEM((IDX_BLK, CHUNK), jnp.int32),
            pltpu.VMEM((IDX_BLK, CHUNK), jnp.int32),
            pltpu.VMEM((IDX_BLK, CHUNK), jnp.int32),
            pltpu.VMEM((CHUNK, D), jnp.float32),
            pltpu.VMEM((CHUNK, D), jnp.float32),
            pltpu.VMEM((CHUNK, D), jnp.float32),
            pltpu.VMEM((CHUNK, D), jnp.float32),
            pltpu.VMEM_SHARED((N_PAD, D), jnp.float32),
            pltpu.SemaphoreType.DMA,
            pltpu.SemaphoreType.DMA,
            pltpu.SemaphoreType.DMA,
            pltpu.SemaphoreType.DMA,
            pltpu.SemaphoreType.DMA,
            pltpu.SemaphoreType.DMA,
            pltpu.SemaphoreType.DMA,
            pltpu.SemaphoreType.DMA,
            pltpu.SemaphoreType.DMA,
            pltpu.SemaphoreType.DMA,
            pltpu.SemaphoreType.DMA,
            pltpu.SemaphoreType.DMA,
            pltpu.SemaphoreType.DMA,
            pltpu.SemaphoreType.DMA,
        ],
    )
    def k(y_hbm, src_hbm, dst_hbm, z_hbm, out_hbm,
          src0, src1, src2, dst0, dst1, dst2,
          rows0, rows1, rows2, rows3, acc_s,
          sg0, sg1, sg2, sg3, ss0, ss1, ss2, ss3,
          si0, si1, si2, di0, di1, di2):
        c = lax.axis_index("c")
        s = lax.axis_index("s")
        wid = c * NS + s
        srcb, dstb = [src0, src1, src2], [dst0, dst1, dst2]
        si, di = [si0, si1, si2], [di0, di1, di2]
        rows = [rows0, rows1, rows2, rows3]
        sg, ss = [sg0, sg1, sg2, sg3], [ss0, ss1, ss2, ss3]

        for k_ in range(NS):
            @pl.when(s == k_)
            def _(k_=k_):
                pltpu.sync_copy(z_hbm.at[pl.ds(k_ * RPT, RPT)],
                                acc_s.at[pl.ds(k_ * RPT, RPT)])

        plsc.subcore_barrier()

        NBUF = 4
        gh = [None] * NBUF
        sh = [None] * NBUF
        gi = [None] * NBUF

        def issue_scatter(jq):
            q = jq % NBUF
            gh[q].wait()
            gh[q] = None
            pb, pr = gi[q]
            sh[q] = pltpu.async_copy(rows[q], acc_s.at[dstb[pb].at[pr]],
                                     ss[q], add=True)

        ihs = [None, None, None]
        ihd = [None, None, None]
        ihs[0] = pltpu.async_copy(src_hbm.at[wid, 0], srcb[0], si[0])
        ihd[0] = pltpu.async_copy(dst_hbm.at[wid, 0], dstb[0], di[0])
        for t in range(OUTER):
            bi = t % 3
            ihs[bi].wait()
            ihd[bi].wait()
            if t + 1 < OUTER:
                nb = (t + 1) % 3
                ihs[nb] = pltpu.async_copy(src_hbm.at[wid, t + 1], srcb[nb],
                                           si[nb])
                ihd[nb] = pltpu.async_copy(dst_hbm.at[wid, t + 1], dstb[nb],
                                           di[nb])
            for r in range(IDX_BLK):
                j = t * IDX_BLK + r
                p = j % NBUF
                if sh[p] is not None:
                    sh[p].wait()
                    sh[p] = None
                gh[p] = pltpu.async_copy(y_hbm.at[srcb[bi].at[r]], rows[p],
                                         sg[p])
                gi[p] = (bi, r)
                if j >= NBUF - 1:
                    issue_scatter(j - (NBUF - 1))
        for jq in range(NCHUNK - NBUF + 1, NCHUNK):
            issue_scatter(jq)
        for p in range(NBUF):
            if sh[p] is not None:
                sh[p].wait()

        plsc.subcore_barrier()

        for k_ in range(NS):
            @pl.when(s == k_)
            def _(k_=k_):
                pltpu.sync_copy(acc_s.at[pl.ds(k_ * RPT, RPT)],
                                out_hbm.at[c, pl.ds(k_ * RPT, RPT)])

    return k(y, src4d, dst4d, zrows)


BR = 400


def _dis_block(deg_ref):
    degsum = deg_ref[0] + deg_ref[1]
    return lax.rsqrt(1.0 + degsum[:, :1])


def _tc_xw(x, W1):
    def body(x_ref, w_ref, o_ref):
        o_ref[...] = jnp.dot(x_ref[...], w_ref[...],
                             preferred_element_type=jnp.float32)

    return pl.pallas_call(
        body,
        grid=(N_NODES // BR,),
        in_specs=[
            pl.BlockSpec((BR, D), lambda i: (i, 0)),
            pl.BlockSpec((D, D), lambda i: (0, 0)),
        ],
        out_specs=pl.BlockSpec((BR, D), lambda i: (i, 0)),
        out_shape=jax.ShapeDtypeStruct((N_NODES, D), jnp.float32),
    )(x, W1)


def _tc_scale(xw, deg):
    def body(xw_ref, deg_ref, y_ref):
        y_ref[...] = xw_ref[...] * _dis_block(deg_ref)

    return pl.pallas_call(
        body,
        grid=(N_NODES // BR,),
        in_specs=[
            pl.BlockSpec((BR, D), lambda i: (i, 0)),
            pl.BlockSpec((2, BR, DEG_LANES), lambda i: (0, i, 0)),
        ],
        out_specs=pl.BlockSpec((BR, D), lambda i: (i, 0)),
        out_shape=jax.ShapeDtypeStruct((N_NODES, D), jnp.float32),
    )(xw, deg)


def _tc_mid(agg, y1, deg, W2, b1):
    def body(agg_ref, y1_ref, deg_ref, w_ref, b_ref, y2_ref):
        dis = _dis_block(deg_ref)
        t = (agg_ref[0] + agg_ref[1] + y1_ref[...]) * dis + b_ref[...]
        h = jnp.where(t >= 0, t, 0.01 * t)
        y2_ref[...] = jnp.dot(h, w_ref[...], preferred_element_type=jnp.float32) * dis

    return pl.pallas_call(
        body,
        grid=(N_NODES // BR,),
        in_specs=[
            pl.BlockSpec((2, BR, D), lambda i: (0, i, 0)),
            pl.BlockSpec((BR, D), lambda i: (i, 0)),
            pl.BlockSpec((2, BR, DEG_LANES), lambda i: (0, i, 0)),
            pl.BlockSpec((D, D), lambda i: (0, 0)),
            pl.BlockSpec((1, D), lambda i: (0, 0)),
        ],
        out_specs=pl.BlockSpec((BR, D), lambda i: (i, 0)),
        out_shape=jax.ShapeDtypeStruct((N_NODES, D), jnp.float32),
    )(agg, y1, deg, W2, b1.reshape(1, D))


def _tc_out(agg, y2, deg, b2):
    def body(agg_ref, y2_ref, deg_ref, b_ref, o_ref):
        dis = _dis_block(deg_ref)
        o_ref[...] = (agg_ref[0] + agg_ref[1] + y2_ref[...]) * dis + b_ref[...]

    return pl.pallas_call(
        body,
        grid=(N_NODES // BR,),
        in_specs=[
            pl.BlockSpec((2, BR, D), lambda i: (0, i, 0)),
            pl.BlockSpec((BR, D), lambda i: (i, 0)),
            pl.BlockSpec((2, BR, DEG_LANES), lambda i: (0, i, 0)),
            pl.BlockSpec((1, D), lambda i: (0, 0)),
        ],
        out_specs=pl.BlockSpec((BR, D), lambda i: (i, 0)),
        out_shape=jax.ShapeDtypeStruct((N_NODES, D), jnp.float32),
    )(agg, y2, deg, b2.reshape(1, D))


def kernel(x, edge_index, W1, b1, W2, b2):
    src3d = edge_index[0].astype(jnp.int32).reshape(NW, OUTER, IDX_BLK, CHUNK)
    dst3d = edge_index[1].astype(jnp.int32).reshape(NW, OUTER, IDX_BLK, CHUNK)

    xw = _tc_xw(x, W1)
    deg = _sc_indeg(dst3d, jnp.zeros((N_PAD, DEG_LANES), jnp.float32))
    zrows = jnp.zeros((N_PAD, D), jnp.float32)
    y1 = _tc_scale(xw, deg)
    agg1 = _sc_agg(y1, src3d, dst3d, zrows)
    y2 = _tc_mid(agg1, y1, deg, W2, b1)
    agg2 = _sc_agg(y2, src3d, dst3d, zrows)
    return _tc_out(agg2, y2, deg, b2)
